# Initial kernel scaffold; baseline (speedup 1.0000x reference)
import jax
import jax.numpy as jnp
from jax.experimental import pallas as pl


def kernel(x, edge_index, W1_l, b1_l, W1_r, W2_l, b2_l, W2_r):
    raise NotImplementedError("write your pallas kernel here")


# speedup vs baseline: 3.1385x; 3.1385x over previous
import functools

import jax
import jax.numpy as jnp
from jax import lax
from jax.experimental import pallas as pl
from jax.experimental.pallas import tpu as pltpu
from jax.experimental.pallas import tpu_sc as plsc

N_NODES = 10000
NPAD = 10240
N_EDGES = 160000
F = 128
NC = 2
NS = 16
CH = 80
CHUNKS = N_EDGES // NS // CH
TR = NPAD // NS
CR = NPAD // (NC * NS)
RB = 1024


def _make_sc_agg():
    mesh = plsc.VectorSubcoreMesh(core_axis_name="c", subcore_axis_name="s",
                                  num_cores=NC, num_subcores=NS)

    @functools.partial(
        pl.kernel,
        out_type=jax.ShapeDtypeStruct((NC * NPAD, F), jnp.float32),
        mesh=mesh,
        scratch_types=[
            pltpu.VMEM((CH, F), jnp.float32),
            pltpu.VMEM((CH,), jnp.int32),
            pltpu.VMEM((CH,), jnp.int32),
            pltpu.VMEM_SHARED((NPAD, F), jnp.float32),
            pltpu.SemaphoreType.DMA,
        ],
    )
    def sc_agg(xcat, srccat, dst, zero128, aggcat, rows, srcb, dstb,
               agg_sh, sem):
        cid = lax.axis_index("c")
        sid = lax.axis_index("s")
        r0 = sid * TR

        pltpu.sync_copy(zero128, agg_sh.at[pl.ds(r0, TR)])
        plsc.subcore_barrier()

        def chunk(k, _):
            base = sid * (N_EDGES // NS) + k * CH
            pltpu.sync_copy(srccat.at[pl.ds(cid * N_EDGES + base, CH)], srcb)
            pltpu.sync_copy(dst.at[pl.ds(base, CH)], dstb)
            pltpu.async_copy(xcat.at[srcb], rows, sem).wait()
            pltpu.sync_copy(rows, agg_sh.at[dstb], add=True)
            return 0

        lax.fori_loop(0, CHUNKS, chunk, 0)
        plsc.subcore_barrier()

        pltpu.sync_copy(agg_sh.at[pl.ds(r0, TR)],
                        aggcat.at[pl.ds(cid * NPAD + r0, TR)])

    return sc_agg


def _make_sc_cnt():
    mesh = plsc.VectorSubcoreMesh(core_axis_name="c", subcore_axis_name="s",
                                  num_cores=NC, num_subcores=NS)

    @functools.partial(
        pl.kernel,
        out_type=jax.ShapeDtypeStruct((NPAD, F), jnp.float32),
        mesh=mesh,
        scratch_types=[
            pltpu.VMEM((CH, F), jnp.float32),
            pltpu.VMEM((CH,), jnp.int32),
            pltpu.VMEM_SHARED((NPAD, F), jnp.float32),
        ],
    )
    def sc_cnt(dst, zero128, ones128, cnt_out, onesb, dstb, cnt_sh):
        cid = lax.axis_index("c")
        sid = lax.axis_index("s")
        r0 = sid * TR

        pltpu.sync_copy(zero128, cnt_sh.at[pl.ds(r0, TR)])
        pltpu.sync_copy(ones128, onesb)
        plsc.subcore_barrier()

        def chunk(k, _):
            base = sid * (N_EDGES // NS) + k * CH
            pltpu.sync_copy(dst.at[pl.ds(base, CH)], dstb)
            pltpu.sync_copy(onesb, cnt_sh.at[dstb], add=True)
            return 0

        lax.fori_loop(0, CHUNKS, chunk, 0)
        plsc.subcore_barrier()

        c0 = cid * (NPAD // NC) + sid * CR
        pltpu.sync_copy(cnt_sh.at[pl.ds(c0, CR)], cnt_out.at[pl.ds(c0, CR)])

    return sc_cnt


def _dense_body(relu, a0, a1, cnt, z0, z1, wlt, wlb, wrt, wrb, b, out):
    inv = 1.0 / jnp.maximum(cnt[:, 0:1], 1.0)
    acc = jnp.dot(a0[...] * inv, wlt[...], preferred_element_type=jnp.float32)
    acc = acc + jnp.dot(a1[...] * inv, wlb[...],
                        preferred_element_type=jnp.float32)
    acc = acc + jnp.dot(z0[...], wrt[...], preferred_element_type=jnp.float32)
    acc = acc + jnp.dot(z1[...], wrb[...], preferred_element_type=jnp.float32)
    acc = acc + b[...]
    if relu:
        acc = jnp.maximum(acc, 0.0)
    out[...] = acc


def _dense(relu: bool, stacked_out: bool):
    nb = NPAD // RB
    if stacked_out:
        out_shape = jax.ShapeDtypeStruct((NC * NPAD, F), jnp.float32)
        out_spec = pl.BlockSpec((RB, F), lambda i, j: (j * (NPAD // RB) + i, 0))
    else:
        out_shape = jax.ShapeDtypeStruct((N_NODES, 2 * F), jnp.float32)
        out_spec = pl.BlockSpec((RB, F), lambda i, j: (i, j))

    call = pl.pallas_call(
        functools.partial(_dense_body, relu),
        grid=(nb, 2),
        in_specs=[
            pl.BlockSpec((RB, F), lambda i, j: (i, 0)),
            pl.BlockSpec((RB, F), lambda i, j: (i + NPAD // RB, 0)),
            pl.BlockSpec((RB, F), lambda i, j: (i, 0)),
            pl.BlockSpec((RB, F), lambda i, j: (i, 0)),
            pl.BlockSpec((RB, F), lambda i, j: (i + NPAD // RB, 0)),
            pl.BlockSpec((F, F), lambda i, j: (0, j)),
            pl.BlockSpec((F, F), lambda i, j: (0, j)),
            pl.BlockSpec((F, F), lambda i, j: (0, j)),
            pl.BlockSpec((F, F), lambda i, j: (0, j)),
            pl.BlockSpec((1, F), lambda i, j: (0, j)),
        ],
        out_specs=out_spec,
        out_shape=out_shape,
    )

    def run(aggcat, cnt, zcat, W_l, W_r, b):
        return call(aggcat, aggcat, cnt, zcat, zcat,
                    W_l[:F], W_l[F:], W_r[:F], W_r[F:], b.reshape(1, 2 * F))
    return run


_make_sc_agg = functools.lru_cache(None)(_make_sc_agg)
_make_sc_cnt = functools.lru_cache(None)(_make_sc_cnt)
_dense = functools.lru_cache(None)(_dense)


def kernel(x, edge_index, W1_l, b1_l, W1_r, W2_l, b2_l, W2_r):
    src = edge_index[0]
    dst = edge_index[1]
    pad = jnp.zeros((NPAD - N_NODES, F), jnp.float32)
    xcat = jnp.concatenate([x[:, :F], pad, x[:, F:], pad], axis=0)
    srccat = jnp.concatenate([src, src + NPAD])
    zero128 = jnp.zeros((TR, F), jnp.float32)
    ones128 = jnp.ones((CH, F), jnp.float32)

    cnt = _make_sc_cnt()(dst, zero128, ones128)
    agg1 = _make_sc_agg()(xcat, srccat, dst, zero128)
    hcat = _dense(True, True)(agg1, cnt, xcat, W1_l, W1_r, b1_l)
    agg2 = _make_sc_agg()(hcat, srccat, dst, zero128)
    out = _dense(False, False)(agg2, cnt, hcat, W2_l, W2_r, b2_l)
    return out


# speedup vs baseline: 5.6216x; 1.7912x over previous
import functools

import jax
import jax.numpy as jnp
from jax import lax
from jax.experimental import pallas as pl
from jax.experimental.pallas import tpu as pltpu
from jax.experimental.pallas import tpu_sc as plsc

N_NODES = 10000
NPAD = 10240
N_EDGES = 160000
F = 128
NC = 2
NS = 16
CH = 40
U = 5
CHUNKS = N_EDGES // NS // CH
CC = 40
CCHUNKS = N_EDGES // (NC * NS) // CC
TR = NPAD // NS
RB = 1024


def _make_sc_agg():
    mesh = plsc.VectorSubcoreMesh(core_axis_name="c", subcore_axis_name="s",
                                  num_cores=NC, num_subcores=NS)

    @functools.partial(
        pl.kernel,
        out_type=jax.ShapeDtypeStruct((NC * NPAD, F), jnp.float32),
        mesh=mesh,
        scratch_types=(
            [pltpu.VMEM((CH, F), jnp.float32) for _ in range(U)]
            + [pltpu.VMEM((CH,), jnp.int32) for _ in range(U)]
            + [pltpu.VMEM((CH,), jnp.int32) for _ in range(U)]
            + [
                pltpu.VMEM_SHARED((NPAD, F), jnp.float32),
                pltpu.SemaphoreType.DMA,
                pltpu.SemaphoreType.DMA,
                pltpu.SemaphoreType.DMA,
            ]
        ),
    )
    def sc_agg(xcat, srccat, dst, zero128, aggcat, *scr):
        rows = scr[0:U]
        srcb = scr[U:2 * U]
        dstb = scr[2 * U:3 * U]
        agg_sh, gsem, ssem, dsem = scr[3 * U:]

        cid = lax.axis_index("c")
        sid = lax.axis_index("s")
        r0 = sid * TR

        pltpu.sync_copy(zero128, agg_sh.at[pl.ds(r0, TR)])
        plsc.subcore_barrier()

        sbase = cid * N_EDGES + sid * (N_EDGES // NS)
        dbase = sid * (N_EDGES // NS)

        def body(i, _):
            k0 = i * U
            sd = [pltpu.async_copy(
                      srccat.at[pl.ds(sbase + (k0 + u) * CH, CH)],
                      srcb[u], ssem) for u in range(U)]
            dd = [pltpu.async_copy(
                      dst.at[pl.ds(dbase + (k0 + u) * CH, CH)],
                      dstb[u], dsem) for u in range(U)]
            gd = []
            for u in range(U):
                sd[u].wait()
                gd.append(pltpu.async_copy(xcat.at[srcb[u]], rows[u], gsem))
            for u in range(U):
                gd[u].wait()
                dd[u].wait()
                pltpu.sync_copy(rows[u], agg_sh.at[dstb[u]], add=True)
            return 0

        lax.fori_loop(0, CHUNKS // U, body, 0)
        plsc.subcore_barrier()

        pltpu.sync_copy(agg_sh.at[pl.ds(r0, TR)],
                        aggcat.at[pl.ds(cid * NPAD + r0, TR)])

    return sc_agg


def _make_sc_cnt():
    mesh = plsc.VectorSubcoreMesh(core_axis_name="c", subcore_axis_name="s",
                                  num_cores=NC, num_subcores=NS)

    @functools.partial(
        pl.kernel,
        out_type=jax.ShapeDtypeStruct((NC * NPAD, F), jnp.float32),
        mesh=mesh,
        scratch_types=(
            [pltpu.VMEM((CC, F), jnp.float32)]
            + [pltpu.VMEM((CC,), jnp.int32) for _ in range(U)]
            + [
                pltpu.VMEM_SHARED((NPAD, F), jnp.float32),
                pltpu.SemaphoreType.DMA,
            ]
        ),
    )
    def sc_cnt(dst, zero128, ones128, cnt_out, *scr):
        onesb = scr[0]
        dstb = scr[1:1 + U]
        cnt_sh, dsem = scr[1 + U:]

        cid = lax.axis_index("c")
        sid = lax.axis_index("s")
        wid = cid * NS + sid
        r0 = sid * TR

        pltpu.sync_copy(zero128, cnt_sh.at[pl.ds(r0, TR)])
        pltpu.sync_copy(ones128, onesb)
        plsc.subcore_barrier()

        ebase = wid * (N_EDGES // (NC * NS))

        def body(i, _):
            k0 = i * U
            dd = [pltpu.async_copy(dst.at[pl.ds(ebase + (k0 + u) * CC, CC)],
                                   dstb[u], dsem) for u in range(U)]
            for u in range(U):
                dd[u].wait()
                pltpu.sync_copy(onesb, cnt_sh.at[dstb[u]], add=True)
            return 0

        lax.fori_loop(0, CCHUNKS // U, body, 0)
        plsc.subcore_barrier()

        pltpu.sync_copy(cnt_sh.at[pl.ds(r0, TR)],
                        cnt_out.at[pl.ds(cid * NPAD + r0, TR)])

    return sc_cnt


def _dense_body(relu, a0, a1, c0, c1, z0, z1, wlt, wlb, wrt, wrb, b, out):
    inv = 1.0 / jnp.maximum(c0[:, 0:1] + c1[:, 0:1], 1.0)
    acc = jnp.dot(a0[...] * inv, wlt[...], preferred_element_type=jnp.float32)
    acc = acc + jnp.dot(a1[...] * inv, wlb[...],
                        preferred_element_type=jnp.float32)
    acc = acc + jnp.dot(z0[...], wrt[...], preferred_element_type=jnp.float32)
    acc = acc + jnp.dot(z1[...], wrb[...], preferred_element_type=jnp.float32)
    acc = acc + b[...]
    if relu:
        acc = jnp.maximum(acc, 0.0)
    out[...] = acc


def _dense(relu: bool, stacked_out: bool):
    nb = NPAD // RB
    if stacked_out:
        out_shape = jax.ShapeDtypeStruct((NC * NPAD, F), jnp.float32)
        out_spec = pl.BlockSpec((RB, F), lambda i, j: (j * (NPAD // RB) + i, 0))
    else:
        out_shape = jax.ShapeDtypeStruct((N_NODES, 2 * F), jnp.float32)
        out_spec = pl.BlockSpec((RB, F), lambda i, j: (i, j))

    call = pl.pallas_call(
        functools.partial(_dense_body, relu),
        grid=(nb, 2),
        in_specs=[
            pl.BlockSpec((RB, F), lambda i, j: (i, 0)),
            pl.BlockSpec((RB, F), lambda i, j: (i + NPAD // RB, 0)),
            pl.BlockSpec((RB, F), lambda i, j: (i, 0)),
            pl.BlockSpec((RB, F), lambda i, j: (i + NPAD // RB, 0)),
            pl.BlockSpec((RB, F), lambda i, j: (i, 0)),
            pl.BlockSpec((RB, F), lambda i, j: (i + NPAD // RB, 0)),
            pl.BlockSpec((F, F), lambda i, j: (0, j)),
            pl.BlockSpec((F, F), lambda i, j: (0, j)),
            pl.BlockSpec((F, F), lambda i, j: (0, j)),
            pl.BlockSpec((F, F), lambda i, j: (0, j)),
            pl.BlockSpec((1, F), lambda i, j: (0, j)),
        ],
        out_specs=out_spec,
        out_shape=out_shape,
    )

    def run(aggcat, cnt, zcat, W_l, W_r, b):
        return call(aggcat, aggcat, cnt, cnt, zcat, zcat,
                    W_l[:F], W_l[F:], W_r[:F], W_r[F:], b.reshape(1, 2 * F))
    return run


_make_sc_agg = functools.lru_cache(None)(_make_sc_agg)
_make_sc_cnt = functools.lru_cache(None)(_make_sc_cnt)
_dense = functools.lru_cache(None)(_dense)


def kernel(x, edge_index, W1_l, b1_l, W1_r, W2_l, b2_l, W2_r):
    src = edge_index[0]
    dst = edge_index[1]
    pad = jnp.zeros((NPAD - N_NODES, F), jnp.float32)
    xcat = jnp.concatenate([x[:, :F], pad, x[:, F:], pad], axis=0)
    srccat = jnp.concatenate([src, src + NPAD])
    zero128 = jnp.zeros((TR, F), jnp.float32)
    ones128 = jnp.ones((CC, F), jnp.float32)

    cnt = _make_sc_cnt()(dst, zero128, ones128)
    agg1 = _make_sc_agg()(xcat, srccat, dst, zero128)
    hcat = _dense(True, True)(agg1, cnt, xcat, W1_l, W1_r, b1_l)
    agg2 = _make_sc_agg()(hcat, srccat, dst, zero128)
    out = _dense(False, False)(agg2, cnt, hcat, W2_l, W2_r, b2_l)
    return out
